# Initial kernel scaffold; baseline (speedup 1.0000x reference)
import functools

import numpy as np
import jax
import jax.numpy as jnp
from jax import lax
from jax.experimental import pallas as pl
from jax.experimental.pallas import tpu as pltpu
from jax.experimental.pallas import tpu_sc as plsc

_LENGTH = 4096
_MASK_RATIO = 0.75
_MASK_LEN = round(_LENGTH * _MASK_RATIO)
_NUM_VIS = _LENGTH - _MASK_LEN + 1
_NW = 32
_PER_W = 128
_TAIL_PAD = 8

_INDEX_CACHE = None


def _index_constants():
    global _INDEX_CACHE
    if _INDEX_CACHE is None:
        with jax.ensure_compile_time_eval():
            perm = jax.random.permutation(jax.random.key(42), _LENGTH - 1) + 1
            perm = np.asarray(perm).astype(np.int32)
        mask_idx = perm[:_MASK_LEN]
        vis_idx = np.concatenate([np.zeros((1,), np.int32), perm[_MASK_LEN:]])
        _INDEX_CACHE = (vis_idx, mask_idx)
    return _INDEX_CACHE


@functools.lru_cache(maxsize=None)
def _build_gather(total_rows, d):
    mesh = plsc.VectorSubcoreMesh(core_axis_name="c", subcore_axis_name="s")

    @functools.partial(
        pl.kernel,
        mesh=mesh,
        out_type=jax.ShapeDtypeStruct((total_rows, d), jnp.float32),
        scratch_types=[
            pltpu.VMEM((_PER_W,), jnp.int32),
            pltpu.VMEM((_PER_W, d), jnp.float32),
            pltpu.VMEM((_TAIL_PAD,), jnp.int32),
            pltpu.VMEM((_TAIL_PAD, d), jnp.float32),
            pltpu.SemaphoreType.DMA,
        ],
    )
    def gather_k(x_hbm, idx_hbm, tail_hbm, out_hbm, idx_v, rows_v, tidx_v,
                 trows_v, sem):
        wid = lax.axis_index("s") * 2 + lax.axis_index("c")
        base = wid * _PER_W
        pltpu.sync_copy(idx_hbm.at[wid], idx_v)
        pltpu.async_copy(x_hbm.at[idx_v], rows_v, sem).wait()
        pltpu.sync_copy(rows_v, out_hbm.at[pl.ds(base, _PER_W)])

        n_tail = total_rows - _NW * _PER_W

        @pl.when(wid == 0)
        def _():
            pltpu.sync_copy(tail_hbm, tidx_v)
            pltpu.async_copy(x_hbm.at[tidx_v], trows_v, sem).wait()
            pltpu.sync_copy(trows_v.at[pl.ds(0, n_tail)],
                            out_hbm.at[pl.ds(_NW * _PER_W, n_tail)])

    return gather_k


def kernel(x):
    vis_idx, mask_idx = _index_constants()
    b, length, d = x.shape
    total_rows = b * _NUM_VIS
    g = (np.arange(b, dtype=np.int32)[:, None] * length
         + vis_idx[None, :]).reshape(-1)
    idx_main = g[:_NW * _PER_W].reshape(_NW, _PER_W)
    tail = np.zeros((_TAIL_PAD,), np.int32)
    tail[:total_rows - _NW * _PER_W] = g[_NW * _PER_W:]

    xf = x.reshape(b * length, d)
    out = _build_gather(total_rows, d)(xf, jnp.asarray(idx_main),
                                       jnp.asarray(tail))
    return (out.reshape(b, _NUM_VIS, d),
            jnp.asarray(vis_idx),
            jnp.asarray(mask_idx))


# speedup vs baseline: 2.3988x; 2.3988x over previous
import functools

import numpy as np
import jax
import jax.numpy as jnp
from jax import lax
from jax.experimental import pallas as pl
from jax.experimental.pallas import tpu as pltpu
from jax.experimental.pallas import tpu_sc as plsc

_LENGTH = 4096
_MASK_RATIO = 0.75
_MASK_LEN = round(_LENGTH * _MASK_RATIO)
_NUM_VIS = _LENGTH - 1 - _MASK_LEN + 1
_NW = 32
_PER_W = 128

_INDEX_CACHE = None


def _index_constants():
    global _INDEX_CACHE
    if _INDEX_CACHE is None:
        with jax.ensure_compile_time_eval(), \
                jax.default_device(jax.devices("cpu")[0]):
            perm = jax.random.permutation(jax.random.key(42), _LENGTH - 1) + 1
            perm = np.asarray(perm).astype(np.int32)
        mask_idx = perm[:_MASK_LEN]
        vis_idx = np.concatenate([np.zeros((1,), np.int32), perm[_MASK_LEN:]])
        _INDEX_CACHE = (vis_idx, mask_idx)
    return _INDEX_CACHE


@functools.lru_cache(maxsize=None)
def _build_gather(total_rows, d):
    mesh = plsc.VectorSubcoreMesh(core_axis_name="c", subcore_axis_name="s")

    @functools.partial(
        pl.kernel,
        mesh=mesh,
        out_type=jax.ShapeDtypeStruct((total_rows, d), jnp.float32),
        scratch_types=[
            pltpu.VMEM((_PER_W,), jnp.int32),
            pltpu.VMEM((_PER_W, d), jnp.float32),
            pltpu.SemaphoreType.DMA,
        ],
    )
    def gather_k(x_hbm, idx_hbm, out_hbm, idx_v, rows_v, sem):
        wid = lax.axis_index("s") * 2 + lax.axis_index("c")
        pltpu.sync_copy(idx_hbm.at[wid], idx_v)
        pltpu.async_copy(x_hbm.at[idx_v], rows_v, sem).wait()
        pltpu.sync_copy(rows_v, out_hbm.at[pl.ds(wid * _PER_W, _PER_W)])

    return gather_k


def kernel(x):
    vis_idx, mask_idx = _index_constants()
    b, length, d = x.shape
    total_rows = b * _NUM_VIS
    g = (np.arange(b, dtype=np.int32)[:, None] * length
         + vis_idx[None, :]).reshape(_NW, _PER_W)

    xf = x.reshape(b * length, d)
    out = _build_gather(total_rows, d)(xf, jnp.asarray(g))
    return (out.reshape(b, _NUM_VIS, d),
            jnp.asarray(vis_idx),
            jnp.asarray(mask_idx))


# speedup vs baseline: 2.4067x; 1.0033x over previous
import functools

import numpy as np
import jax
import jax.numpy as jnp
from jax import lax
from jax.experimental import pallas as pl
from jax.experimental.pallas import tpu as pltpu
from jax.experimental.pallas import tpu_sc as plsc

_LENGTH = 4096
_MASK_RATIO = 0.75
_MASK_LEN = round(_LENGTH * _MASK_RATIO)
_NUM_VIS = _LENGTH - 1 - _MASK_LEN + 1
_NW = 32
_PER_W = 128
_NCHUNK = 4
_CH = _PER_W // _NCHUNK

_INDEX_CACHE = None


def _index_constants():
    global _INDEX_CACHE
    if _INDEX_CACHE is None:
        with jax.ensure_compile_time_eval(), \
                jax.default_device(jax.devices("cpu")[0]):
            perm = jax.random.permutation(jax.random.key(42), _LENGTH - 1) + 1
            perm = np.asarray(perm).astype(np.int32)
        mask_idx = perm[:_MASK_LEN]
        vis_idx = np.concatenate([np.zeros((1,), np.int32), perm[_MASK_LEN:]])
        _INDEX_CACHE = (vis_idx, mask_idx)
    return _INDEX_CACHE


@functools.lru_cache(maxsize=None)
def _build_gather(total_rows, d):
    mesh = plsc.VectorSubcoreMesh(core_axis_name="c", subcore_axis_name="s")

    @functools.partial(
        pl.kernel,
        mesh=mesh,
        out_type=jax.ShapeDtypeStruct((total_rows, d), jnp.float32),
        scratch_types=[
            pltpu.VMEM((_NCHUNK, _CH), jnp.int32),
            pltpu.VMEM((_NCHUNK, _CH, d), jnp.float32),
            pltpu.SemaphoreType.DMA((_NCHUNK,)),
            pltpu.SemaphoreType.DMA((_NCHUNK,)),
        ],
    )
    def gather_k(x_hbm, idx_hbm, out_hbm, idx_v, rows_v, gsem, wsem):
        wid = lax.axis_index("s") * 2 + lax.axis_index("c")
        base = wid * _PER_W
        pltpu.sync_copy(idx_hbm.at[wid], idx_v)
        gathers = [
            pltpu.async_copy(x_hbm.at[idx_v.at[j]], rows_v.at[j], gsem.at[j])
            for j in range(_NCHUNK)
        ]
        writes = []
        for j in range(_NCHUNK):
            gathers[j].wait()
            writes.append(
                pltpu.async_copy(rows_v.at[j],
                                 out_hbm.at[pl.ds(base + j * _CH, _CH)],
                                 wsem.at[j]))
        for w in writes:
            w.wait()

    return gather_k


def kernel(x):
    vis_idx, mask_idx = _index_constants()
    b, length, d = x.shape
    total_rows = b * _NUM_VIS
    g = (np.arange(b, dtype=np.int32)[:, None] * length
         + vis_idx[None, :]).reshape(_NW, _NCHUNK, _CH)

    xf = x.reshape(b * length, d)
    out = _build_gather(total_rows, d)(xf, jnp.asarray(g))
    return (out.reshape(b, _NUM_VIS, d),
            jnp.asarray(vis_idx),
            jnp.asarray(mask_idx))
